# baseline (speedup 1.0000x reference)
import functools

import jax
import jax.numpy as jnp
from jax import lax
from jax.experimental import pallas as pl
from jax.experimental.pallas import tpu as pltpu
from jax.experimental.pallas import tpu_sc as plsc

N_VARS = 1_000_000
E = 8_000_000
OUT = 2_000_000
FAN = 4

NC = 2
NS = 16
NW = NC * NS
L = 16

SHIFT = 6
TAB = 2_097_152
R = TAB // 2
RSUB = R // NS
RCP = 32_768
DUM = 8_192

BC = 15_632
NBC = 2
W_PAD = 1_000_008

CH = 3_200
CE = CH * FAN
NCHUNK = OUT // CH
NKMAX = (NCHUNK - 1) // NW + 1

TCROWS, TCCOLS, TCBLK = 2_000, 4_000, 8


def _worker_id():
    return lax.axis_index("s") * NC + lax.axis_index("c")


def _encode_body(w_hbm, xs_hbm, wbuf, ibuf, cbuf, sem):
    wid = _worker_id()
    iota = lax.broadcasted_iota(jnp.int32, (L,), 0)
    start = (N_VARS // NW) * wid // 8 * 8

    def build_chunk(c, _):
        src = start + c * BC
        pltpu.sync_copy(w_hbm.at[pl.ds(src, BC)], wbuf)

        def build_vreg(v, _):
            wv = wbuf[pl.ds(v * L, L)]
            b2 = iota * 2 + v * (2 * L)
            plsc.store_scatter(ibuf, [b2], wv)
            plsc.store_scatter(ibuf, [b2 + 1], 1.0 - wv)
            return 0

        lax.fori_loop(0, BC // L, build_vreg, 0)
        pltpu.sync_copy(ibuf, xs_hbm.at[pl.ds(8 + 2 * src, 2 * BC)])
        return 0

    lax.fori_loop(0, NBC, build_chunk, 0)

    @pl.when(wid == 0)
    def _():
        cbuf[...] = jnp.where(iota == 7, 1.0, 0.0).astype(jnp.float32)
        pltpu.sync_copy(cbuf.at[pl.ds(0, 8)], xs_hbm.at[pl.ds(0, 8)])


def _make_prep_body(shift):
    def body(p_ref, a_ref, b_ref):
        p = p_ref[...]
        dummy = R + (p & (DUM - 1))
        ps = p + shift if shift else p
        a_ref[...] = jnp.where(ps < R, ps, dummy)
        b_ref[...] = jnp.where(ps >= R, ps - R, dummy)

    return body


def _make_layer_body(is_prod, outp):
    ident = 1.0 if is_prod else 0.0

    def body(table_hbm, ia_hbm, ib_hbm, out_hbm, pa_hbm, spm,
             idx0, idx1, g0, g1, o0, o1, pa0, pa1,
             si0, si1, sg0, sg1, so0, so1, sp0, sp1):
        sid = lax.axis_index("s")
        wid = _worker_id()
        iota = lax.broadcasted_iota(jnp.int32, (L,), 0)
        idxb, gb, ob, pab = (idx0, idx1), (g0, g1), (o0, o1), (pa0, pa1)
        sib, sgb, sob, spb = (si0, si1), (sg0, sg1), (so0, so1), (sp0, sp1)

        def cid(j):
            return wid + j * NW

        def stage(half):
            for c in range(RSUB // RCP):
                off = sid * RSUB + c * RCP
                pltpu.sync_copy(table_hbm.at[pl.ds(half * R + off, RCP)],
                                spm.at[pl.ds(off, RCP)])

        stage(0)

        @pl.when(sid == 0)
        def _():
            def fill(v, _):
                g0[pl.ds(v * L, L)] = jnp.full((L,), ident, jnp.float32)
                return 0

            lax.fori_loop(0, DUM // L, fill, 0)
            pltpu.sync_copy(g0.at[pl.ds(0, DUM)], spm.at[pl.ds(R, DUM)])

        plsc.subcore_barrier()

        def run_pass(idx_hbm, dst_hbm, merge):
            def start_idx(j, b):
                pltpu.async_copy(idx_hbm.at[pl.ds(cid(j) * CE, CE)],
                                 idxb[b], sib[b])

            def start_pa(j, b):
                if merge:
                    pltpu.async_copy(pa_hbm.at[pl.ds(cid(j) * CH, CH)],
                                     pab[b], spb[b])

            def start_gather(b):
                pltpu.make_async_copy(idx_hbm.at[pl.ds(0, CE)], idxb[b],
                                      sib[b]).wait()
                pltpu.async_copy(spm.at[idxb[b]], gb[b], sgb[b])

            def reduce_store(j, b):
                def red_vreg(q, _):
                    base = iota * FAN + q * (FAN * L)
                    a0 = plsc.load_gather(gb[b], [base])
                    a1 = plsc.load_gather(gb[b], [base + 1])
                    a2 = plsc.load_gather(gb[b], [base + 2])
                    a3 = plsc.load_gather(gb[b], [base + 3])
                    if is_prod:
                        r = (a0 * a1) * (a2 * a3)
                    else:
                        r = (a0 + a1) + (a2 + a3)
                    if merge:
                        pv = pab[b][pl.ds(q * L, L)]
                        r = r * pv if is_prod else r + pv
                    ob[b][pl.ds(q * L, L)] = r
                    return 0

                if merge:
                    pltpu.make_async_copy(
                        pa_hbm.at[pl.ds(0, CH)], pab[b], spb[b]).wait()
                lax.fori_loop(0, CH // L, red_vreg, 0)
                pltpu.async_copy(ob[b], dst_hbm.at[pl.ds(cid(j) * CH, CH)],
                                 sob[b])

            start_idx(0, 0)
            start_idx(1, 1)
            start_pa(0, 0)
            start_pa(1, 1)
            start_gather(0)

            def pair(t, _):
                for b in (0, 1):
                    j = 2 * t + b
                    nb = 1 - b

                    @pl.when(cid(j + 1) < NCHUNK)
                    def _():
                        start_gather(nb)

                    @pl.when(cid(j) < NCHUNK)
                    def _():
                        pltpu.make_async_copy(spm.at[idxb[b]], gb[b],
                                              sgb[b]).wait()

                    @pl.when(cid(j + 2) < NCHUNK)
                    def _():
                        start_idx(j + 2, b)

                    @pl.when(jnp.logical_and(j >= 2, cid(j) < NCHUNK))
                    def _():
                        pltpu.make_async_copy(
                            ob[b], dst_hbm.at[pl.ds(0, CH)], sob[b]).wait()

                    @pl.when(cid(j) < NCHUNK)
                    def _():
                        reduce_store(j, b)

                    @pl.when(cid(j + 2) < NCHUNK)
                    def _():
                        start_pa(j + 2, b)

                return 0

            lax.fori_loop(0, NKMAX // 2, pair, 0)

            for b in (0, 1):
                pltpu.make_async_copy(ob[b], dst_hbm.at[pl.ds(0, CH)],
                                      sob[b]).wait()

        run_pass(ia_hbm, pa_hbm, False)

        plsc.subcore_barrier()
        stage(1)
        plsc.subcore_barrier()

        run_pass(ib_hbm, out_hbm, True)

    return body


@functools.cache
def _build_calls():
    mesh = plsc.VectorSubcoreMesh(core_axis_name="c", subcore_axis_name="s")
    params = pltpu.CompilerParams(needs_layout_passes=False)
    encode = pl.kernel(
        _encode_body,
        out_type=jax.ShapeDtypeStruct((TAB,), jnp.float32),
        mesh=mesh,
        compiler_params=params,
        scratch_types=[
            pltpu.VMEM((BC,), jnp.float32),
            pltpu.VMEM((2 * BC,), jnp.float32),
            pltpu.VMEM((L,), jnp.float32),
            pltpu.SemaphoreType.DMA,
        ],
    )

    def make_prep(shift):
        return pl.pallas_call(
            _make_prep_body(shift),
            out_shape=[
                jax.ShapeDtypeStruct((TCROWS, TCCOLS), jnp.int32),
                jax.ShapeDtypeStruct((TCROWS, TCCOLS), jnp.int32),
            ],
            grid=(TCROWS // TCBLK,),
            in_specs=[pl.BlockSpec((TCBLK, TCCOLS), lambda i: (i, 0))],
            out_specs=[pl.BlockSpec((TCBLK, TCCOLS), lambda i: (i, 0))] * 2,
        )

    def make_layer(is_prod, outp):
        return pl.kernel(
            _make_layer_body(is_prod, outp),
            out_type=(
                jax.ShapeDtypeStruct((outp,), jnp.float32),
                jax.ShapeDtypeStruct((OUT,), jnp.float32),
            ),
            mesh=mesh,
            compiler_params=params,
            scratch_types=[
                pltpu.VMEM_SHARED((R + DUM,), jnp.float32),
                pltpu.VMEM((CE,), jnp.int32),
                pltpu.VMEM((CE,), jnp.int32),
                pltpu.VMEM((CE,), jnp.float32),
                pltpu.VMEM((CE,), jnp.float32),
                pltpu.VMEM((CH,), jnp.float32),
                pltpu.VMEM((CH,), jnp.float32),
                pltpu.VMEM((CH,), jnp.float32),
                pltpu.VMEM((CH,), jnp.float32),
            ] + [pltpu.SemaphoreType.DMA] * 8,
        )

    return (encode, make_prep(SHIFT), make_prep(0),
            make_layer(True, TAB), make_layer(False, OUT))


def kernel(weights, ptrs0, ptrs1):
    encode, prep0, prep1, layer0, layer1 = _build_calls()
    w_pad = jnp.concatenate(
        [weights, jnp.zeros((W_PAD - N_VARS,), jnp.float32)])
    xs = encode(w_pad)
    ia0, ib0 = prep0(ptrs0.reshape(TCROWS, TCCOLS))
    ia1, ib1 = prep1(ptrs1.reshape(TCROWS, TCCOLS))
    y0, _ = layer0(xs, ia0.reshape(E), ib0.reshape(E))
    y1, _ = layer1(y0, ia1.reshape(E), ib1.reshape(E))
    return y1


# speedup vs baseline: 1.7766x; 1.7766x over previous
import functools

import jax
import jax.numpy as jnp
from jax import lax
from jax.experimental import pallas as pl
from jax.experimental.pallas import tpu as pltpu
from jax.experimental.pallas import tpu_sc as plsc

N_VARS = 1_000_000
E = 8_000_000
OUT = 2_000_000
FAN = 4

NC = 2
NS = 16
NW = NC * NS
L = 16

SHIFT = 6
TAB = 2_097_152
R = TAB // 2
RSUB = R // NS
RCP = 32_768
DUM = 8_192

BC = 15_632
NBC = 2
W_PAD = 1_000_008

CH = 3_200
CE = CH * FAN
NCHUNK = OUT // CH
NKMAX = (NCHUNK - 1) // NW + 1

def _worker_id():
    return lax.axis_index("s") * NC + lax.axis_index("c")


def _encode_body(w_hbm, xs_hbm, wbuf, ibuf, cbuf, sem):
    wid = _worker_id()
    iota = lax.broadcasted_iota(jnp.int32, (L,), 0)
    start = (N_VARS // NW) * wid // 8 * 8

    def build_chunk(c, _):
        src = start + c * BC
        pltpu.sync_copy(w_hbm.at[pl.ds(src, BC)], wbuf)

        def build_vreg(v, _):
            wv = wbuf[pl.ds(v * L, L)]
            b2 = iota * 2 + v * (2 * L)
            plsc.store_scatter(ibuf, [b2], wv)
            plsc.store_scatter(ibuf, [b2 + 1], 1.0 - wv)
            return 0

        lax.fori_loop(0, BC // L, build_vreg, 0)
        pltpu.sync_copy(ibuf, xs_hbm.at[pl.ds(8 + 2 * src, 2 * BC)])
        return 0

    lax.fori_loop(0, NBC, build_chunk, 0)

    @pl.when(wid == 0)
    def _():
        cbuf[...] = jnp.where(iota == 7, 1.0, 0.0).astype(jnp.float32)
        pltpu.sync_copy(cbuf.at[pl.ds(0, 8)], xs_hbm.at[pl.ds(0, 8)])


def _make_layer_body(is_prod, shift, outp):
    ident = 1.0 if is_prod else 0.0

    def body(table_hbm, p_hbm, out_hbm, pa_hbm, spm,
             idx0, idx1, g0, g1, o0, o1, pa0, pa1,
             si0, si1, sg0, sg1, so0, so1, sp0, sp1):
        sid = lax.axis_index("s")
        wid = _worker_id()
        iota = lax.broadcasted_iota(jnp.int32, (L,), 0)
        idxb, gb, ob, pab = (idx0, idx1), (g0, g1), (o0, o1), (pa0, pa1)
        sib, sgb, sob, spb = (si0, si1), (sg0, sg1), (so0, so1), (sp0, sp1)

        def cid(j):
            return wid + j * NW

        def stage(half):
            for c in range(RSUB // RCP):
                off = sid * RSUB + c * RCP
                pltpu.sync_copy(table_hbm.at[pl.ds(half * R + off, RCP)],
                                spm.at[pl.ds(off, RCP)])

        stage(0)

        @pl.when(sid == 0)
        def _():
            def fill(v, _):
                g0[pl.ds(v * L, L)] = jnp.full((L,), ident, jnp.float32)
                return 0

            lax.fori_loop(0, DUM // L, fill, 0)
            pltpu.sync_copy(g0.at[pl.ds(0, DUM)], spm.at[pl.ds(R, DUM)])

        plsc.subcore_barrier()

        def run_pass(half, dst_hbm, merge):
            def start_idx(j, b):
                pltpu.async_copy(p_hbm.at[pl.ds(cid(j) * CE, CE)],
                                 idxb[b], sib[b])

            def start_pa(j, b):
                if merge:
                    pltpu.async_copy(pa_hbm.at[pl.ds(cid(j) * CH, CH)],
                                     pab[b], spb[b])

            def map_idx(b):
                def map8(v, _):
                    for u in range(8):
                        sl = pl.ds((v * 8 + u) * L, L)
                        p = idxb[b][sl]
                        ps = p + shift if shift else p
                        dummy = (p & (DUM - 1)) + R
                        if half == 0:
                            val = jnp.where(ps < R, ps, dummy)
                        else:
                            val = jnp.where(ps >= R, ps - R, dummy)
                        idxb[b][sl] = val
                    return 0

                lax.fori_loop(0, CE // L // 8, map8, 0)

            def start_gather(b):
                pltpu.make_async_copy(p_hbm.at[pl.ds(0, CE)], idxb[b],
                                      sib[b]).wait()
                map_idx(b)
                pltpu.async_copy(spm.at[idxb[b]], gb[b], sgb[b])

            def reduce_store(j, b):
                def red_vreg(q, _):
                    base = iota * FAN + q * (FAN * L)
                    a0 = plsc.load_gather(gb[b], [base])
                    a1 = plsc.load_gather(gb[b], [base + 1])
                    a2 = plsc.load_gather(gb[b], [base + 2])
                    a3 = plsc.load_gather(gb[b], [base + 3])
                    if is_prod:
                        r = (a0 * a1) * (a2 * a3)
                    else:
                        r = (a0 + a1) + (a2 + a3)
                    if merge:
                        pv = pab[b][pl.ds(q * L, L)]
                        r = r * pv if is_prod else r + pv
                    ob[b][pl.ds(q * L, L)] = r
                    return 0

                if merge:
                    pltpu.make_async_copy(
                        pa_hbm.at[pl.ds(0, CH)], pab[b], spb[b]).wait()
                lax.fori_loop(0, CH // L, red_vreg, 0)
                pltpu.async_copy(ob[b], dst_hbm.at[pl.ds(cid(j) * CH, CH)],
                                 sob[b])

            start_idx(0, 0)
            start_idx(1, 1)
            start_pa(0, 0)
            start_pa(1, 1)
            start_gather(0)

            def pair(t, _):
                for b in (0, 1):
                    j = 2 * t + b
                    nb = 1 - b

                    @pl.when(cid(j + 1) < NCHUNK)
                    def _():
                        start_gather(nb)

                    @pl.when(cid(j) < NCHUNK)
                    def _():
                        pltpu.make_async_copy(spm.at[idxb[b]], gb[b],
                                              sgb[b]).wait()

                    @pl.when(cid(j + 2) < NCHUNK)
                    def _():
                        start_idx(j + 2, b)

                    @pl.when(jnp.logical_and(j >= 2, cid(j) < NCHUNK))
                    def _():
                        pltpu.make_async_copy(
                            ob[b], dst_hbm.at[pl.ds(0, CH)], sob[b]).wait()

                    @pl.when(cid(j) < NCHUNK)
                    def _():
                        reduce_store(j, b)

                    @pl.when(cid(j + 2) < NCHUNK)
                    def _():
                        start_pa(j + 2, b)

                return 0

            lax.fori_loop(0, NKMAX // 2, pair, 0)

            for b in (0, 1):
                pltpu.make_async_copy(ob[b], dst_hbm.at[pl.ds(0, CH)],
                                      sob[b]).wait()

        run_pass(0, pa_hbm, False)

        plsc.subcore_barrier()
        stage(1)
        plsc.subcore_barrier()

        run_pass(1, out_hbm, True)

    return body


@functools.cache
def _build_calls():
    mesh = plsc.VectorSubcoreMesh(core_axis_name="c", subcore_axis_name="s")
    params = pltpu.CompilerParams(needs_layout_passes=False)
    encode = pl.kernel(
        _encode_body,
        out_type=jax.ShapeDtypeStruct((TAB,), jnp.float32),
        mesh=mesh,
        compiler_params=params,
        scratch_types=[
            pltpu.VMEM((BC,), jnp.float32),
            pltpu.VMEM((2 * BC,), jnp.float32),
            pltpu.VMEM((L,), jnp.float32),
            pltpu.SemaphoreType.DMA,
        ],
    )

    def make_layer(is_prod, shift, outp):
        return pl.kernel(
            _make_layer_body(is_prod, shift, outp),
            out_type=(
                jax.ShapeDtypeStruct((outp,), jnp.float32),
                jax.ShapeDtypeStruct((OUT,), jnp.float32),
            ),
            mesh=mesh,
            compiler_params=params,
            scratch_types=[
                pltpu.VMEM_SHARED((R + DUM,), jnp.float32),
                pltpu.VMEM((CE,), jnp.int32),
                pltpu.VMEM((CE,), jnp.int32),
                pltpu.VMEM((CE,), jnp.float32),
                pltpu.VMEM((CE,), jnp.float32),
                pltpu.VMEM((CH,), jnp.float32),
                pltpu.VMEM((CH,), jnp.float32),
                pltpu.VMEM((CH,), jnp.float32),
                pltpu.VMEM((CH,), jnp.float32),
            ] + [pltpu.SemaphoreType.DMA] * 8,
        )

    return (encode, make_layer(True, SHIFT, TAB), make_layer(False, 0, OUT))


def kernel(weights, ptrs0, ptrs1):
    encode, layer0, layer1 = _build_calls()
    w_pad = jnp.concatenate(
        [weights, jnp.zeros((W_PAD - N_VARS,), jnp.float32)])
    xs = encode(w_pad)
    y0, _ = layer0(xs, ptrs0)
    y1, _ = layer1(y0, ptrs1)
    return y1


# speedup vs baseline: 1.8309x; 1.0306x over previous
import functools

import jax
import jax.numpy as jnp
from jax import lax
from jax.experimental import pallas as pl
from jax.experimental.pallas import tpu as pltpu
from jax.experimental.pallas import tpu_sc as plsc

N_VARS = 1_000_000
E = 8_000_000
OUT = 2_000_000
FAN = 4

NC = 2
NS = 16
NW = NC * NS
L = 16

SHIFT = 6
TAB = 2_097_152
R = TAB // 2
RSUB = R // NS
RCP = 32_768
DUM = 8_192

BC = 15_632
NBC = 2
W_PAD = 1_000_008

CH = 3_200
CE = CH * FAN
NCHUNK = OUT // CH
NKMAX = (NCHUNK - 1) // NW + 1

def _worker_id():
    return lax.axis_index("s") * NC + lax.axis_index("c")


def _encode_body(w_hbm, xs_hbm, wbuf, ibuf, cbuf, sem):
    wid = _worker_id()
    iota = lax.broadcasted_iota(jnp.int32, (L,), 0)
    start = (N_VARS // NW) * wid // 8 * 8

    def build_chunk(c, _):
        src = start + c * BC
        pltpu.sync_copy(w_hbm.at[pl.ds(src, BC)], wbuf)

        def build_vreg(v, _):
            wv = wbuf[pl.ds(v * L, L)]
            b2 = iota * 2 + v * (2 * L)
            plsc.store_scatter(ibuf, [b2], wv)
            plsc.store_scatter(ibuf, [b2 + 1], 1.0 - wv)
            return 0

        lax.fori_loop(0, BC // L, build_vreg, 0)
        pltpu.sync_copy(ibuf, xs_hbm.at[pl.ds(8 + 2 * src, 2 * BC)])
        return 0

    lax.fori_loop(0, NBC, build_chunk, 0)

    @pl.when(wid == 0)
    def _():
        cbuf[...] = jnp.where(iota == 7, 1.0, 0.0).astype(jnp.float32)
        pltpu.sync_copy(cbuf.at[pl.ds(0, 8)], xs_hbm.at[pl.ds(0, 8)])


def _make_layer_body(is_prod, shift, outp):
    ident = 1.0 if is_prod else 0.0

    def body(table_hbm, p_hbm, out_hbm, pa_hbm, spm,
             idx0, idx1, g0, g1, o0, o1, pa0, pa1,
             si0, si1, sg0, sg1, so0, so1, sp0, sp1, sh0, sh1):
        sid = lax.axis_index("s")
        wid = _worker_id()
        iota = lax.broadcasted_iota(jnp.int32, (L,), 0)
        idxb, gb, ob, pab = (idx0, idx1), (g0, g1), (o0, o1), (pa0, pa1)
        sib, sgb, sob, spb = (si0, si1), (sg0, sg1), (so0, so1), (sp0, sp1)
        shb = (sh0, sh1)
        HE = CE // 2

        def cid(j):
            return wid + j * NW

        def stage(half):
            for c in range(RSUB // RCP):
                off = sid * RSUB + c * RCP
                pltpu.sync_copy(table_hbm.at[pl.ds(half * R + off, RCP)],
                                spm.at[pl.ds(off, RCP)])

        stage(0)

        @pl.when(sid == 0)
        def _():
            def fill(v, _):
                g0[pl.ds(v * L, L)] = jnp.full((L,), ident, jnp.float32)
                return 0

            lax.fori_loop(0, DUM // L, fill, 0)
            pltpu.sync_copy(g0.at[pl.ds(0, DUM)], spm.at[pl.ds(R, DUM)])

        plsc.subcore_barrier()

        def run_pass(half, dst_hbm, merge):
            def start_idx(j, b):
                pltpu.async_copy(p_hbm.at[pl.ds(cid(j) * CE, CE)],
                                 idxb[b], sib[b])

            def start_pa(j, b):
                if merge:
                    pltpu.async_copy(pa_hbm.at[pl.ds(cid(j) * CH, CH)],
                                     pab[b], spb[b])

            def map_idx(b):
                def map8(v, _):
                    for u in range(8):
                        sl = pl.ds((v * 8 + u) * L, L)
                        p = idxb[b][sl]
                        ps = p + shift if shift else p
                        dummy = (p & (DUM - 1)) + R
                        if half == 0:
                            val = jnp.where(ps < R, ps, dummy)
                        else:
                            val = jnp.where(ps >= R, ps - R, dummy)
                        idxb[b][sl] = val
                    return 0

                lax.fori_loop(0, CE // L // 8, map8, 0)

            def start_gather(b):
                pltpu.make_async_copy(p_hbm.at[pl.ds(0, CE)], idxb[b],
                                      sib[b]).wait()
                map_idx(b)
                pltpu.async_copy(spm.at[idxb[b].at[pl.ds(0, HE)]],
                                 gb[b].at[pl.ds(0, HE)], sgb[b])
                pltpu.async_copy(spm.at[idxb[b].at[pl.ds(HE, HE)]],
                                 gb[b].at[pl.ds(HE, HE)], shb[b])

            def reduce_store(j, b):
                def red_vreg(q, _):
                    base = iota * FAN + q * (FAN * L)
                    a0 = plsc.load_gather(gb[b], [base])
                    a1 = plsc.load_gather(gb[b], [base + 1])
                    a2 = plsc.load_gather(gb[b], [base + 2])
                    a3 = plsc.load_gather(gb[b], [base + 3])
                    if is_prod:
                        r = (a0 * a1) * (a2 * a3)
                    else:
                        r = (a0 + a1) + (a2 + a3)
                    if merge:
                        pv = pab[b][pl.ds(q * L, L)]
                        r = r * pv if is_prod else r + pv
                    ob[b][pl.ds(q * L, L)] = r
                    return 0

                if merge:
                    pltpu.make_async_copy(
                        pa_hbm.at[pl.ds(0, CH)], pab[b], spb[b]).wait()
                lax.fori_loop(0, CH // L, red_vreg, 0)
                pltpu.async_copy(ob[b], dst_hbm.at[pl.ds(cid(j) * CH, CH)],
                                 sob[b])

            start_idx(0, 0)
            start_idx(1, 1)
            start_pa(0, 0)
            start_pa(1, 1)
            start_gather(0)

            def pair(t, _):
                for b in (0, 1):
                    j = 2 * t + b
                    nb = 1 - b

                    @pl.when(cid(j + 1) < NCHUNK)
                    def _():
                        start_gather(nb)

                    @pl.when(cid(j) < NCHUNK)
                    def _():
                        pltpu.make_async_copy(
                            spm.at[idxb[b].at[pl.ds(0, HE)]],
                            gb[b].at[pl.ds(0, HE)], sgb[b]).wait()
                        pltpu.make_async_copy(
                            spm.at[idxb[b].at[pl.ds(HE, HE)]],
                            gb[b].at[pl.ds(HE, HE)], shb[b]).wait()

                    @pl.when(cid(j + 2) < NCHUNK)
                    def _():
                        start_idx(j + 2, b)

                    @pl.when(jnp.logical_and(j >= 2, cid(j) < NCHUNK))
                    def _():
                        pltpu.make_async_copy(
                            ob[b], dst_hbm.at[pl.ds(0, CH)], sob[b]).wait()

                    @pl.when(cid(j) < NCHUNK)
                    def _():
                        reduce_store(j, b)

                    @pl.when(cid(j + 2) < NCHUNK)
                    def _():
                        start_pa(j + 2, b)

                return 0

            lax.fori_loop(0, NKMAX // 2, pair, 0)

            for b in (0, 1):
                pltpu.make_async_copy(ob[b], dst_hbm.at[pl.ds(0, CH)],
                                      sob[b]).wait()

        run_pass(0, pa_hbm, False)

        plsc.subcore_barrier()
        stage(1)
        plsc.subcore_barrier()

        run_pass(1, out_hbm, True)

    return body


@functools.cache
def _build_calls():
    mesh = plsc.VectorSubcoreMesh(core_axis_name="c", subcore_axis_name="s")
    params = pltpu.CompilerParams(needs_layout_passes=False)
    encode = pl.kernel(
        _encode_body,
        out_type=jax.ShapeDtypeStruct((TAB,), jnp.float32),
        mesh=mesh,
        compiler_params=params,
        scratch_types=[
            pltpu.VMEM((BC,), jnp.float32),
            pltpu.VMEM((2 * BC,), jnp.float32),
            pltpu.VMEM((L,), jnp.float32),
            pltpu.SemaphoreType.DMA,
        ],
    )

    def make_layer(is_prod, shift, outp):
        return pl.kernel(
            _make_layer_body(is_prod, shift, outp),
            out_type=(
                jax.ShapeDtypeStruct((outp,), jnp.float32),
                jax.ShapeDtypeStruct((OUT,), jnp.float32),
            ),
            mesh=mesh,
            compiler_params=params,
            scratch_types=[
                pltpu.VMEM_SHARED((R + DUM,), jnp.float32),
                pltpu.VMEM((CE,), jnp.int32),
                pltpu.VMEM((CE,), jnp.int32),
                pltpu.VMEM((CE,), jnp.float32),
                pltpu.VMEM((CE,), jnp.float32),
                pltpu.VMEM((CH,), jnp.float32),
                pltpu.VMEM((CH,), jnp.float32),
                pltpu.VMEM((CH,), jnp.float32),
                pltpu.VMEM((CH,), jnp.float32),
            ] + [pltpu.SemaphoreType.DMA] * 10,
        )

    return (encode, make_layer(True, SHIFT, TAB), make_layer(False, 0, OUT))


def kernel(weights, ptrs0, ptrs1):
    encode, layer0, layer1 = _build_calls()
    w_pad = jnp.concatenate(
        [weights, jnp.zeros((W_PAD - N_VARS,), jnp.float32)])
    xs = encode(w_pad)
    y0, _ = layer0(xs, ptrs0)
    y1, _ = layer1(y0, ptrs1)
    return y1


# speedup vs baseline: 1.8316x; 1.0004x over previous
import functools

import jax
import jax.numpy as jnp
from jax import lax
from jax.experimental import pallas as pl
from jax.experimental.pallas import tpu as pltpu
from jax.experimental.pallas import tpu_sc as plsc

N_VARS = 1_000_000
OUT = 2_000_000
FAN = 4

NC = 2
NS = 16
NW = NC * NS
L = 16

SHIFT = 6
TAB = 2_097_152
R = TAB // 2
RSUB = R // NS
RCP = 32_768
DUM = 8_192

BC = 15_632
NBC = 2
W_PAD = 1_000_008

CH = 3_200
CE = CH * FAN
NCHUNK = OUT // CH
NKMAX = (NCHUNK - 1) // NW + 1

def _worker_id():
    return lax.axis_index("s") * NC + lax.axis_index("c")


def _encode_body(w_hbm, xs_hbm, wbuf, ibuf, cbuf, sem):
    wid = _worker_id()
    iota = lax.broadcasted_iota(jnp.int32, (L,), 0)
    start = (N_VARS // NW) * wid // 8 * 8

    def build_chunk(c, _):
        src = start + c * BC
        pltpu.sync_copy(w_hbm.at[pl.ds(src, BC)], wbuf)

        def build_vreg(v, _):
            wv = wbuf[pl.ds(v * L, L)]
            b2 = iota * 2 + v * (2 * L)
            plsc.store_scatter(ibuf, [b2], wv)
            plsc.store_scatter(ibuf, [b2 + 1], 1.0 - wv)
            return 0

        lax.fori_loop(0, BC // L, build_vreg, 0)
        pltpu.sync_copy(ibuf, xs_hbm.at[pl.ds(8 + 2 * src, 2 * BC)])
        return 0

    lax.fori_loop(0, NBC, build_chunk, 0)

    @pl.when(wid == 0)
    def _():
        cbuf[...] = jnp.where(iota == 7, 1.0, 0.0).astype(jnp.float32)
        pltpu.sync_copy(cbuf.at[pl.ds(0, 8)], xs_hbm.at[pl.ds(0, 8)])


def _make_layer_body(is_prod, shift, outp):
    ident = 1.0 if is_prod else 0.0

    def body(table_hbm, p_hbm, out_hbm, pa_hbm, spm,
             idx0, idx1, g0, g1, o0, o1, pa0, pa1,
             si0, si1, sg0, sg1, so0, so1, sp0, sp1, sh0, sh1):
        sid = lax.axis_index("s")
        wid = _worker_id()
        iota = lax.broadcasted_iota(jnp.int32, (L,), 0)
        idxb, gb, ob, pab = (idx0, idx1), (g0, g1), (o0, o1), (pa0, pa1)
        sib, sgb, sob, spb = (si0, si1), (sg0, sg1), (so0, so1), (sp0, sp1)
        shb = (sh0, sh1)
        HE = CE // 2

        def cid(j):
            return wid + j * NW

        def stage(half):
            for c in range(RSUB // RCP):
                off = sid * RSUB + c * RCP
                pltpu.sync_copy(table_hbm.at[pl.ds(half * R + off, RCP)],
                                spm.at[pl.ds(off, RCP)])

        stage(0)

        @pl.when(sid == 0)
        def _():
            def fill(v, _):
                g0[pl.ds(v * L, L)] = jnp.full((L,), ident, jnp.float32)
                return 0

            lax.fori_loop(0, DUM // L, fill, 0)
            pltpu.sync_copy(g0.at[pl.ds(0, DUM)], spm.at[pl.ds(R, DUM)])

        plsc.subcore_barrier()

        def run_pass(half, dst_hbm, merge):
            def start_idx(j, b):
                pltpu.async_copy(p_hbm.at[pl.ds(cid(j) * CE, CE)],
                                 idxb[b], sib[b])

            def start_pa(j, b):
                if merge:
                    pltpu.async_copy(pa_hbm.at[pl.ds(cid(j) * CH, CH)],
                                     pab[b], spb[b])

            def map_idx(b):
                def map8(v, _):
                    for u in range(8):
                        sl = pl.ds((v * 8 + u) * L, L)
                        p = idxb[b][sl]
                        ps = p + shift if shift else p
                        dummy = (p & (DUM - 1)) + R
                        if half == 0:
                            val = jnp.where(ps < R, ps, dummy)
                        else:
                            val = jnp.where(ps >= R, ps - R, dummy)
                        idxb[b][sl] = val
                    return 0

                lax.fori_loop(0, CE // L // 8, map8, 0)

            def start_gather(b):
                pltpu.make_async_copy(p_hbm.at[pl.ds(0, CE)], idxb[b],
                                      sib[b]).wait()
                map_idx(b)
                pltpu.async_copy(spm.at[idxb[b].at[pl.ds(0, HE)]],
                                 gb[b].at[pl.ds(0, HE)], sgb[b])
                pltpu.async_copy(spm.at[idxb[b].at[pl.ds(HE, HE)]],
                                 gb[b].at[pl.ds(HE, HE)], shb[b])

            def reduce_store(j, b):
                def red_vreg(q, _):
                    base = iota * FAN + q * (FAN * L)
                    a0 = plsc.load_gather(gb[b], [base])
                    a1 = plsc.load_gather(gb[b], [base + 1])
                    a2 = plsc.load_gather(gb[b], [base + 2])
                    a3 = plsc.load_gather(gb[b], [base + 3])
                    if is_prod:
                        r = (a0 * a1) * (a2 * a3)
                    else:
                        r = (a0 + a1) + (a2 + a3)
                    if merge:
                        pv = pab[b][pl.ds(q * L, L)]
                        r = r * pv if is_prod else r + pv
                    ob[b][pl.ds(q * L, L)] = r
                    return 0

                if merge:
                    pltpu.make_async_copy(
                        pa_hbm.at[pl.ds(0, CH)], pab[b], spb[b]).wait()
                lax.fori_loop(0, CH // L, red_vreg, 0)
                pltpu.async_copy(ob[b], dst_hbm.at[pl.ds(cid(j) * CH, CH)],
                                 sob[b])

            start_idx(0, 0)
            start_idx(1, 1)
            start_pa(0, 0)
            start_pa(1, 1)
            start_gather(0)

            def pair(t, _):
                for b in (0, 1):
                    j = 2 * t + b
                    nb = 1 - b

                    @pl.when(cid(j + 1) < NCHUNK)
                    def _():
                        start_gather(nb)

                    @pl.when(cid(j) < NCHUNK)
                    def _():
                        pltpu.make_async_copy(
                            spm.at[idxb[b].at[pl.ds(0, HE)]],
                            gb[b].at[pl.ds(0, HE)], sgb[b]).wait()
                        pltpu.make_async_copy(
                            spm.at[idxb[b].at[pl.ds(HE, HE)]],
                            gb[b].at[pl.ds(HE, HE)], shb[b]).wait()

                    @pl.when(cid(j + 2) < NCHUNK)
                    def _():
                        start_idx(j + 2, b)

                    @pl.when(jnp.logical_and(j >= 2, cid(j) < NCHUNK))
                    def _():
                        pltpu.make_async_copy(
                            ob[b], dst_hbm.at[pl.ds(0, CH)], sob[b]).wait()

                    @pl.when(cid(j) < NCHUNK)
                    def _():
                        reduce_store(j, b)

                    @pl.when(cid(j + 2) < NCHUNK)
                    def _():
                        start_pa(j + 2, b)

                return 0

            lax.fori_loop(0, NKMAX // 2, pair, 0)

            for b in (0, 1):
                pltpu.make_async_copy(ob[b], dst_hbm.at[pl.ds(0, CH)],
                                      sob[b]).wait()

        run_pass(0, pa_hbm, False)

        plsc.subcore_barrier()
        stage(1)
        plsc.subcore_barrier()

        run_pass(1, out_hbm, True)

    return body


@functools.cache
def _build_calls():
    mesh = plsc.VectorSubcoreMesh(core_axis_name="c", subcore_axis_name="s")
    params = pltpu.CompilerParams(needs_layout_passes=False)
    encode = pl.kernel(
        _encode_body,
        out_type=jax.ShapeDtypeStruct((TAB,), jnp.float32),
        mesh=mesh,
        compiler_params=params,
        scratch_types=[
            pltpu.VMEM((BC,), jnp.float32),
            pltpu.VMEM((2 * BC,), jnp.float32),
            pltpu.VMEM((L,), jnp.float32),
            pltpu.SemaphoreType.DMA,
        ],
    )

    def make_layer(is_prod, shift, outp):
        return pl.kernel(
            _make_layer_body(is_prod, shift, outp),
            out_type=(
                jax.ShapeDtypeStruct((outp,), jnp.float32),
                jax.ShapeDtypeStruct((OUT,), jnp.float32),
            ),
            mesh=mesh,
            compiler_params=params,
            scratch_types=[
                pltpu.VMEM_SHARED((R + DUM,), jnp.float32),
                pltpu.VMEM((CE,), jnp.int32),
                pltpu.VMEM((CE,), jnp.int32),
                pltpu.VMEM((CE,), jnp.float32),
                pltpu.VMEM((CE,), jnp.float32),
                pltpu.VMEM((CH,), jnp.float32),
                pltpu.VMEM((CH,), jnp.float32),
                pltpu.VMEM((CH,), jnp.float32),
                pltpu.VMEM((CH,), jnp.float32),
            ] + [pltpu.SemaphoreType.DMA] * 10,
        )

    return (encode, make_layer(True, SHIFT, TAB), make_layer(False, 0, OUT))


def kernel(weights, ptrs0, ptrs1):
    encode, layer0, layer1 = _build_calls()
    w_pad = jnp.concatenate(
        [weights, jnp.zeros((W_PAD - N_VARS,), jnp.float32)])
    xs = encode(w_pad)
    y0, _ = layer0(xs, ptrs0)
    y1, _ = layer1(y0, ptrs1)
    return y1
